# baseline (speedup 1.0000x reference)
import functools

import jax
import jax.numpy as jnp
from jax import lax
from jax.experimental import pallas as pl
from jax.experimental.pallas import tpu as pltpu, tpu_sc as plsc

_N = 10000
_E = 320000
_D = 128

_MESH = plsc.VectorSubcoreMesh(core_axis_name="c", subcore_axis_name="s")
_NC = _MESH.num_cores
_NS = _MESH.num_subcores
_NW = _NC * _NS
_CH = 80
_NCHT = 128
_EPT = _NCHT * _CH
_EPAD = _NW * _EPT
_NBUF = 2
_NITER = _NCHT // _NBUF
_NPAD = 10240
_ZROWS = _NPAD // _NS


def _unpack_edges(src_v, dst_v):
    def urow(r, _):
        def ucol(c, _):
            v = src_v[r, pl.ds(c * 16, 16)]
            dst_v[r, pl.ds(c * 16, 16)] = lax.shift_right_logical(v, 14)
            src_v[r, pl.ds(c * 16, 16)] = v & 0x3FFF
            return 0
        return lax.fori_loop(0, _CH // 16, ucol, 0)
    lax.fori_loop(0, _NCHT, urow, 0)


@functools.partial(
    pl.kernel,
    out_type=jax.ShapeDtypeStruct((_NC, _NPAD, 16), jnp.float32),
    mesh=_MESH,
    compiler_params=pltpu.CompilerParams(use_tc_tiling_on_sc=False),
    scratch_types=[
        pltpu.VMEM((_NCHT, _CH), jnp.int32),
        pltpu.VMEM((_NCHT, _CH), jnp.int32),
        pltpu.VMEM((_CH, 16), jnp.float32),
        pltpu.VMEM((_CH, 16), jnp.float32),
        pltpu.VMEM_SHARED((_NPAD, 16), jnp.float32),
        pltpu.SemaphoreType.DMA,
        pltpu.SemaphoreType.DMA,
    ],
)
def _deg_kernel(epk, out, src_v, dst_v, po, pi, table, s0, s1):
    cid = lax.axis_index("c")
    sid = lax.axis_index("s")
    wid = sid * _NC + cid
    pltpu.sync_copy(epk.at[wid], src_v)

    zero16 = jnp.zeros((16,), jnp.float32)

    def zfill(i, _):
        po[i, pl.ds(0, 16)] = zero16
        return 0

    lax.fori_loop(0, _CH, zfill, 0)
    for z in range(_ZROWS // _CH):
        pltpu.sync_copy(po, table.at[pl.ds(sid * _ZROWS + z * _CH, _CH)])

    lane = lax.iota(jnp.int32, 16)
    po_vec = jnp.where(lane < 8, 1.0, 0.0).astype(jnp.float32)
    pi_vec = jnp.where(lane >= 8, 1.0, 0.0).astype(jnp.float32)

    def fill(i, _):
        po[i, pl.ds(0, 16)] = po_vec
        pi[i, pl.ds(0, 16)] = pi_vec
        return 0

    lax.fori_loop(0, _CH, fill, 0)
    _unpack_edges(src_v, dst_v)
    plsc.subcore_barrier()

    def chunk(j, _):
        @pl.when(j >= 2)
        def _():
            pltpu.make_async_copy(po, table.at[src_v.at[j - 2]], s0).wait()
            pltpu.make_async_copy(pi, table.at[dst_v.at[j - 2]], s1).wait()

        pltpu.async_copy(po, table.at[src_v.at[j]], s0, add=True)
        pltpu.async_copy(pi, table.at[dst_v.at[j]], s1, add=True)
        return 0

    lax.fori_loop(0, _NCHT, chunk, 0)
    for jl in (_NCHT - 2, _NCHT - 1):
        pltpu.make_async_copy(po, table.at[src_v.at[jl]], s0).wait()
        pltpu.make_async_copy(pi, table.at[dst_v.at[jl]], s1).wait()
    plsc.subcore_barrier()
    pltpu.sync_copy(
        table.at[pl.ds(sid * _ZROWS, _ZROWS)],
        out.at[cid, pl.ds(sid * _ZROWS, _ZROWS)],
    )


@functools.partial(
    pl.kernel,
    out_type=jax.ShapeDtypeStruct((_NC, _NPAD, _D), jnp.float32),
    mesh=_MESH,
    compiler_params=pltpu.CompilerParams(use_tc_tiling_on_sc=False),
    scratch_types=[
        pltpu.VMEM((_NCHT, _CH), jnp.int32),
        pltpu.VMEM((_NCHT, _CH), jnp.int32),
        pltpu.VMEM((_NBUF, _CH, _D), jnp.float32),
        pltpu.VMEM_SHARED((_NPAD, _D), jnp.float32),
        pltpu.SemaphoreType.DMA,
        pltpu.SemaphoreType.DMA,
    ],
)
def _agg_kernel(h, epk, out, src_v, dst_v, bufs, acc, g0, g1):
    gs = (g0, g1)
    cid = lax.axis_index("c")
    sid = lax.axis_index("s")
    wid = sid * _NC + cid
    pltpu.sync_copy(epk.at[wid], src_v)

    zero16 = jnp.zeros((16,), jnp.float32)

    def zrow(i, _):
        def zcol(c, _):
            bufs[0, i, pl.ds(c * 16, 16)] = zero16
            return 0
        return lax.fori_loop(0, _D // 16, zcol, 0)

    lax.fori_loop(0, _CH, zrow, 0)
    for z in range(_ZROWS // _CH):
        pltpu.sync_copy(
            bufs.at[0], acc.at[pl.ds(sid * _ZROWS + z * _CH, _CH)])

    _unpack_edges(src_v, dst_v)
    plsc.subcore_barrier()

    def body(i, _):
        for k in range(_NBUF):
            j = _NBUF * i + k
            pltpu.sync_copy(h.at[src_v.at[j]], bufs.at[k])
            pltpu.sync_copy(bufs.at[k], acc.at[dst_v.at[j]], add=True)
        return 0

    lax.fori_loop(0, _NITER, body, 0)
    plsc.subcore_barrier()
    pltpu.sync_copy(
        acc.at[pl.ds(sid * _ZROWS, _ZROWS)],
        out.at[cid, pl.ds(sid * _ZROWS, _ZROWS)],
    )


def _norm_body(degs_ref, x_ref, h_ref, nin_ref, nout_ref):
    d_out = degs_ref[0, :, 0:1] + degs_ref[1, :, 0:1]
    d_in = degs_ref[0, :, 8:9] + degs_ref[1, :, 8:9]
    n_out = jnp.where(d_out > 0, lax.rsqrt(jnp.maximum(d_out, 1.0)), 0.0)
    n_in = jnp.where(d_in > 0, lax.rsqrt(jnp.maximum(d_in, 1.0)), 0.0)
    h_ref[...] = x_ref[...] * n_out
    nin_ref[...] = n_in
    nout_ref[...] = n_out


def _layer_body(p_ref, nin_ref, nout_ref, w_ref, b_ref, o_ref, *, relu):
    a = (p_ref[0] + p_ref[1]) * nin_ref[...]
    o = jnp.dot(a, w_ref[...], preferred_element_type=jnp.float32) + b_ref[...]
    if relu:
        o = jnp.maximum(o, 0.0) * nout_ref[...]
    o_ref[...] = o


_BN = 1024


def _norm_call(degs, x):
    return pl.pallas_call(
        _norm_body,
        grid=(_NPAD // _BN,),
        in_specs=[
            pl.BlockSpec((_NC, _BN, 16), lambda i: (0, i, 0)),
            pl.BlockSpec((_BN, _D), lambda i: (i, 0)),
        ],
        out_specs=[
            pl.BlockSpec((_BN, _D), lambda i: (i, 0)),
            pl.BlockSpec((_BN, 1), lambda i: (i, 0)),
            pl.BlockSpec((_BN, 1), lambda i: (i, 0)),
        ],
        out_shape=[
            jax.ShapeDtypeStruct((_NPAD, _D), jnp.float32),
            jax.ShapeDtypeStruct((_NPAD, 1), jnp.float32),
            jax.ShapeDtypeStruct((_NPAD, 1), jnp.float32),
        ],
    )(degs, x)


def _layer_call(p, n_in, n_out, wm, bm, relu):
    return pl.pallas_call(
        functools.partial(_layer_body, relu=relu),
        grid=(_NPAD // _BN,),
        in_specs=[
            pl.BlockSpec((_NC, _BN, _D), lambda i: (0, i, 0)),
            pl.BlockSpec((_BN, 1), lambda i: (i, 0)),
            pl.BlockSpec((_BN, 1), lambda i: (i, 0)),
            pl.BlockSpec((_D, _D), lambda i: (0, 0)),
            pl.BlockSpec((1, _D), lambda i: (0, 0)),
        ],
        out_specs=pl.BlockSpec((_BN, _D), lambda i: (i, 0)),
        out_shape=jax.ShapeDtypeStruct((_NPAD, _D), jnp.float32),
    )(p, n_in, n_out, wm, bm)


def kernel(x, edge_index, W1, b1, W2, b2):
    pad = _EPAD - _E
    ei = jnp.concatenate(
        [edge_index, jnp.full((2, pad), _N, jnp.int32)], axis=1)
    epk = (ei[0] | (ei[1] << 14)).reshape(_NW, _NCHT, _CH)
    x_pad = jnp.pad(x, ((0, _NPAD - _N), (0, 0)))

    degs = _deg_kernel(epk)
    h1, n_in, n_out = _norm_call(degs, x_pad)
    p1 = _agg_kernel(h1, epk)
    h2 = _layer_call(p1, n_in, n_out, W1, b1.reshape(1, _D), relu=True)
    p2 = _agg_kernel(h2, epk)
    out = _layer_call(p2, n_in, n_out, W2, b2.reshape(1, _D), relu=False)
    return out[:_N]


# speedup vs baseline: 2.6465x; 2.6465x over previous
import functools

import jax
import jax.numpy as jnp
from jax import lax
from jax.experimental import pallas as pl
from jax.experimental.pallas import tpu as pltpu, tpu_sc as plsc

_N = 10000
_E = 320000
_D = 128

_MESH = plsc.VectorSubcoreMesh(core_axis_name="c", subcore_axis_name="s")
_NC = _MESH.num_cores
_NS = _MESH.num_subcores
_NW = _NC * _NS
_CH = 80
_NCHT = 128
_EPT = _NCHT * _CH
_EPAD = _NW * _EPT
_NBUF = 2
_NITER = _NCHT // _NBUF
_NPAD = 10240
_ZROWS = _NPAD // _NS


def _unpack_edges(src_v, dst_v):
    def urow(r, _):
        def ucol(c, _):
            v = src_v[r, pl.ds(c * 16, 16)]
            dst_v[r, pl.ds(c * 16, 16)] = lax.shift_right_logical(v, 14)
            src_v[r, pl.ds(c * 16, 16)] = v & 0x3FFF
            return 0
        return lax.fori_loop(0, _CH // 16, ucol, 0)
    lax.fori_loop(0, _NCHT, urow, 0)


@functools.partial(
    pl.kernel,
    out_type=jax.ShapeDtypeStruct((_NC, _NPAD, 16), jnp.float32),
    mesh=_MESH,
    compiler_params=pltpu.CompilerParams(use_tc_tiling_on_sc=False),
    scratch_types=[
        pltpu.VMEM((_NCHT, _CH), jnp.int32),
        pltpu.VMEM((_NCHT, _CH), jnp.int32),
        pltpu.VMEM((_CH, 16), jnp.float32),
        pltpu.VMEM((_CH, 16), jnp.float32),
        pltpu.VMEM_SHARED((_NPAD, 16), jnp.float32),
        pltpu.SemaphoreType.DMA,
        pltpu.SemaphoreType.DMA,
    ],
)
def _deg_kernel(epk, out, src_v, dst_v, po, pi, table, s0, s1):
    cid = lax.axis_index("c")
    sid = lax.axis_index("s")
    wid = sid * _NC + cid
    pltpu.sync_copy(epk.at[wid], src_v)

    zero16 = jnp.zeros((16,), jnp.float32)

    def zfill(i, _):
        po[i, pl.ds(0, 16)] = zero16
        return 0

    lax.fori_loop(0, _CH, zfill, 0)
    for z in range(_ZROWS // _CH):
        pltpu.sync_copy(po, table.at[pl.ds(sid * _ZROWS + z * _CH, _CH)])

    lane = lax.iota(jnp.int32, 16)
    po_vec = jnp.where(lane < 8, 1.0, 0.0).astype(jnp.float32)
    pi_vec = jnp.where(lane >= 8, 1.0, 0.0).astype(jnp.float32)

    def fill(i, _):
        po[i, pl.ds(0, 16)] = po_vec
        pi[i, pl.ds(0, 16)] = pi_vec
        return 0

    lax.fori_loop(0, _CH, fill, 0)
    _unpack_edges(src_v, dst_v)
    plsc.subcore_barrier()

    def chunk(j, _):
        @pl.when(j >= 2)
        def _():
            pltpu.make_async_copy(po, table.at[src_v.at[j - 2]], s0).wait()
            pltpu.make_async_copy(pi, table.at[dst_v.at[j - 2]], s1).wait()

        pltpu.async_copy(po, table.at[src_v.at[j]], s0, add=True)
        pltpu.async_copy(pi, table.at[dst_v.at[j]], s1, add=True)
        return 0

    lax.fori_loop(0, _NCHT, chunk, 0)
    for jl in (_NCHT - 2, _NCHT - 1):
        pltpu.make_async_copy(po, table.at[src_v.at[jl]], s0).wait()
        pltpu.make_async_copy(pi, table.at[dst_v.at[jl]], s1).wait()
    plsc.subcore_barrier()
    pltpu.sync_copy(
        table.at[pl.ds(sid * _ZROWS, _ZROWS)],
        out.at[cid, pl.ds(sid * _ZROWS, _ZROWS)],
    )


@functools.partial(
    pl.kernel,
    out_type=jax.ShapeDtypeStruct((_NC, _NPAD, _D), jnp.float32),
    mesh=_MESH,
    compiler_params=pltpu.CompilerParams(use_tc_tiling_on_sc=False),
    scratch_types=[
        pltpu.VMEM((_NCHT, _CH), jnp.int32),
        pltpu.VMEM((_NCHT, _CH), jnp.int32),
        pltpu.VMEM((_NBUF, _CH, _D), jnp.float32),
        pltpu.VMEM_SHARED((_NPAD, _D), jnp.float32),
        pltpu.SemaphoreType.DMA,
        pltpu.SemaphoreType.DMA,
    ],
)
def _agg_kernel(h, epk, out, src_v, dst_v, bufs, acc, g0, g1):
    gs = (g0, g1)
    cid = lax.axis_index("c")
    sid = lax.axis_index("s")
    wid = sid * _NC + cid
    pltpu.sync_copy(epk.at[wid], src_v)

    zero16 = jnp.zeros((16,), jnp.float32)

    def zrow(i, _):
        def zcol(c, _):
            bufs[0, i, pl.ds(c * 16, 16)] = zero16
            return 0
        return lax.fori_loop(0, _D // 16, zcol, 0)

    lax.fori_loop(0, _CH, zrow, 0)
    for z in range(_ZROWS // _CH):
        pltpu.sync_copy(
            bufs.at[0], acc.at[pl.ds(sid * _ZROWS + z * _CH, _CH)])

    _unpack_edges(src_v, dst_v)
    plsc.subcore_barrier()

    def body(i, _):
        for k in range(_NBUF):
            j = _NBUF * i + k
            pltpu.sync_copy(h.at[src_v.at[j]], bufs.at[k])
            pltpu.sync_copy(bufs.at[k], acc.at[dst_v.at[j]], add=True)
        return 0

    lax.fori_loop(0, _NITER, body, 0)
    plsc.subcore_barrier()
    pltpu.sync_copy(
        acc.at[pl.ds(sid * _ZROWS, _ZROWS)],
        out.at[cid, pl.ds(sid * _ZROWS, _ZROWS)],
    )


def _norm_body(degs_ref, x_ref, h_ref, nin_ref, nout_ref):
    d_out = degs_ref[0, :, 0:1] + degs_ref[1, :, 0:1]
    d_in = degs_ref[0, :, 8:9] + degs_ref[1, :, 8:9]
    n_out = jnp.where(d_out > 0, lax.rsqrt(jnp.maximum(d_out, 1.0)), 0.0)
    n_in = jnp.where(d_in > 0, lax.rsqrt(jnp.maximum(d_in, 1.0)), 0.0)
    h_ref[...] = x_ref[...] * n_out
    nin_ref[...] = n_in
    nout_ref[...] = n_out


def _layer_body(p_ref, nin_ref, nout_ref, w_ref, b_ref, o_ref, *, relu):
    a = (p_ref[0] + p_ref[1]) * nin_ref[...]
    o = jnp.dot(a, w_ref[...], preferred_element_type=jnp.float32) + b_ref[...]
    if relu:
        o = jnp.maximum(o, 0.0) * nout_ref[...]
    o_ref[...] = o


_BN = 1024


def _norm_call(degs, x):
    return pl.pallas_call(
        _norm_body,
        grid=(_NPAD // _BN,),
        in_specs=[
            pl.BlockSpec((_NC, _BN, 16), lambda i: (0, i, 0)),
            pl.BlockSpec((_BN, _D), lambda i: (i, 0)),
        ],
        out_specs=[
            pl.BlockSpec((_BN, _D), lambda i: (i, 0)),
            pl.BlockSpec((_BN, 1), lambda i: (i, 0)),
            pl.BlockSpec((_BN, 1), lambda i: (i, 0)),
        ],
        out_shape=[
            jax.ShapeDtypeStruct((_NPAD, _D), jnp.float32),
            jax.ShapeDtypeStruct((_NPAD, 1), jnp.float32),
            jax.ShapeDtypeStruct((_NPAD, 1), jnp.float32),
        ],
    )(degs, x)


def _layer_call(p, n_in, n_out, wm, bm, relu):
    return pl.pallas_call(
        functools.partial(_layer_body, relu=relu),
        grid=(_NPAD // _BN,),
        in_specs=[
            pl.BlockSpec((_NC, _BN, _D), lambda i: (0, i, 0)),
            pl.BlockSpec((_BN, 1), lambda i: (i, 0)),
            pl.BlockSpec((_BN, 1), lambda i: (i, 0)),
            pl.BlockSpec((_D, _D), lambda i: (0, 0)),
            pl.BlockSpec((1, _D), lambda i: (0, 0)),
        ],
        out_specs=pl.BlockSpec((_BN, _D), lambda i: (i, 0)),
        out_shape=jax.ShapeDtypeStruct((_NPAD, _D), jnp.float32),
    )(p, n_in, n_out, wm, bm)


def kernel(x, edge_index, W1, b1, W2, b2):
    pad = _EPAD - _E
    padnode = _N + (jnp.arange(pad, dtype=jnp.int32) % (_NPAD - _N))
    ei = jnp.concatenate(
        [edge_index, jnp.stack([padnode, padnode])], axis=1)
    epk = (ei[0] | (ei[1] << 14)).reshape(_NW, _NCHT, _CH)
    x_pad = jnp.pad(x, ((0, _NPAD - _N), (0, 0)))

    degs = _deg_kernel(epk)
    h1, n_in, n_out = _norm_call(degs, x_pad)
    p1 = _agg_kernel(h1, epk)
    h2 = _layer_call(p1, n_in, n_out, W1, b1.reshape(1, _D), relu=True)
    p2 = _agg_kernel(h2, epk)
    out = _layer_call(p2, n_in, n_out, W2, b2.reshape(1, _D), relu=False)
    return out[:_N]


# speedup vs baseline: 4.0636x; 1.5355x over previous
import functools

import jax
import jax.numpy as jnp
from jax import lax
from jax.experimental import pallas as pl
from jax.experimental.pallas import tpu as pltpu, tpu_sc as plsc

_N = 10000
_E = 320000
_D = 128

_MESH = plsc.VectorSubcoreMesh(core_axis_name="c", subcore_axis_name="s")
_NC = _MESH.num_cores
_NS = _MESH.num_subcores
_NW = _NC * _NS
_CH = 80
_NCHT = 128
_EPT = _NCHT * _CH
_EPAD = _NW * _EPT
_NBUF = 2
_NITER = _NCHT // _NBUF
_NPAD = 10240
_ZROWS = _NPAD // _NS


def _unpack_edges(src_v, dst_v):
    def urow(r, _):
        def ucol(c, _):
            v = src_v[r, pl.ds(c * 16, 16)]
            dst_v[r, pl.ds(c * 16, 16)] = lax.shift_right_logical(v, 14)
            src_v[r, pl.ds(c * 16, 16)] = v & 0x3FFF
            return 0
        return lax.fori_loop(0, _CH // 16, ucol, 0)
    lax.fori_loop(0, _NCHT, urow, 0)


@functools.partial(
    pl.kernel,
    out_type=jax.ShapeDtypeStruct((_NC, _NPAD, 16), jnp.float32),
    mesh=_MESH,
    compiler_params=pltpu.CompilerParams(use_tc_tiling_on_sc=False),
    scratch_types=[
        pltpu.VMEM((_NCHT, _CH), jnp.int32),
        pltpu.VMEM((_NCHT, _CH), jnp.int32),
        pltpu.VMEM((_CH, 16), jnp.float32),
        pltpu.VMEM((_CH, 16), jnp.float32),
        pltpu.VMEM_SHARED((_NPAD, 16), jnp.float32),
        pltpu.SemaphoreType.DMA,
        pltpu.SemaphoreType.DMA,
    ],
)
def _deg_kernel(epk, out, src_v, dst_v, po, pi, table, s0, s1):
    cid = lax.axis_index("c")
    sid = lax.axis_index("s")
    wid = sid * _NC + cid
    pltpu.sync_copy(epk.at[wid], src_v)

    zero16 = jnp.zeros((16,), jnp.float32)

    def zfill(i, _):
        po[i, pl.ds(0, 16)] = zero16
        return 0

    lax.fori_loop(0, _CH, zfill, 0)
    for z in range(_ZROWS // _CH):
        pltpu.sync_copy(po, table.at[pl.ds(sid * _ZROWS + z * _CH, _CH)])

    lane = lax.iota(jnp.int32, 16)
    po_vec = jnp.where(lane < 8, 1.0, 0.0).astype(jnp.float32)
    pi_vec = jnp.where(lane >= 8, 1.0, 0.0).astype(jnp.float32)

    def fill(i, _):
        po[i, pl.ds(0, 16)] = po_vec
        pi[i, pl.ds(0, 16)] = pi_vec
        return 0

    lax.fori_loop(0, _CH, fill, 0)
    _unpack_edges(src_v, dst_v)
    plsc.subcore_barrier()

    def chunk(j, _):
        @pl.when(j >= 2)
        def _():
            pltpu.make_async_copy(po, table.at[src_v.at[j - 2]], s0).wait()
            pltpu.make_async_copy(pi, table.at[dst_v.at[j - 2]], s1).wait()

        pltpu.async_copy(po, table.at[src_v.at[j]], s0, add=True)
        pltpu.async_copy(pi, table.at[dst_v.at[j]], s1, add=True)
        return 0

    lax.fori_loop(0, _NCHT, chunk, 0)
    for jl in (_NCHT - 2, _NCHT - 1):
        pltpu.make_async_copy(po, table.at[src_v.at[jl]], s0).wait()
        pltpu.make_async_copy(pi, table.at[dst_v.at[jl]], s1).wait()
    plsc.subcore_barrier()
    pltpu.sync_copy(
        table.at[pl.ds(sid * _ZROWS, _ZROWS)],
        out.at[cid, pl.ds(sid * _ZROWS, _ZROWS)],
    )


@functools.partial(
    pl.kernel,
    out_type=jax.ShapeDtypeStruct((_NC, _NPAD, _D), jnp.float32),
    mesh=_MESH,
    compiler_params=pltpu.CompilerParams(use_tc_tiling_on_sc=False),
    scratch_types=[
        pltpu.VMEM((_NCHT, _CH), jnp.int32),
        pltpu.VMEM((_NCHT, _CH), jnp.int32),
        pltpu.VMEM((_NBUF, _CH, _D), jnp.float32),
        pltpu.VMEM_SHARED((_NPAD, _D), jnp.float32),
        pltpu.SemaphoreType.DMA,
        pltpu.SemaphoreType.DMA,
    ],
)
def _agg_kernel(h, epk, out, src_v, dst_v, bufs, acc, g0, g1):
    gs = (g0, g1)
    cid = lax.axis_index("c")
    sid = lax.axis_index("s")
    wid = sid * _NC + cid
    pltpu.sync_copy(epk.at[wid], src_v)

    zero16 = jnp.zeros((16,), jnp.float32)

    def zrow(i, _):
        def zcol(c, _):
            bufs[0, i, pl.ds(c * 16, 16)] = zero16
            return 0
        return lax.fori_loop(0, _D // 16, zcol, 0)

    lax.fori_loop(0, _CH, zrow, 0)
    for z in range(_ZROWS // _CH):
        pltpu.sync_copy(
            bufs.at[0], acc.at[pl.ds(sid * _ZROWS + z * _CH, _CH)])

    _unpack_edges(src_v, dst_v)
    plsc.subcore_barrier()

    for k in range(_NBUF):
        pltpu.async_copy(h.at[src_v.at[k]], bufs.at[k], gs[k])

    def body(i, _):
        for k in range(_NBUF):
            j = _NBUF * i + k
            pltpu.make_async_copy(
                h.at[src_v.at[j]], bufs.at[k], gs[k]).wait()
            pltpu.sync_copy(bufs.at[k], acc.at[dst_v.at[j]], add=True)

            @pl.when(i < _NITER - 1)
            def _():
                pltpu.async_copy(
                    h.at[src_v.at[j + _NBUF]], bufs.at[k], gs[k])

        return 0

    lax.fori_loop(0, _NITER, body, 0)
    plsc.subcore_barrier()
    pltpu.sync_copy(
        acc.at[pl.ds(sid * _ZROWS, _ZROWS)],
        out.at[cid, pl.ds(sid * _ZROWS, _ZROWS)],
    )


def _norm_body(degs_ref, x_ref, h_ref, nin_ref, nout_ref):
    d_out = degs_ref[0, :, 0:1] + degs_ref[1, :, 0:1]
    d_in = degs_ref[0, :, 8:9] + degs_ref[1, :, 8:9]
    n_out = jnp.where(d_out > 0, lax.rsqrt(jnp.maximum(d_out, 1.0)), 0.0)
    n_in = jnp.where(d_in > 0, lax.rsqrt(jnp.maximum(d_in, 1.0)), 0.0)
    h_ref[...] = x_ref[...] * n_out
    nin_ref[...] = n_in
    nout_ref[...] = n_out


def _layer_body(p_ref, nin_ref, nout_ref, w_ref, b_ref, o_ref, *, relu):
    a = (p_ref[0] + p_ref[1]) * nin_ref[...]
    o = jnp.dot(a, w_ref[...], preferred_element_type=jnp.float32) + b_ref[...]
    if relu:
        o = jnp.maximum(o, 0.0) * nout_ref[...]
    o_ref[...] = o


_BN = 1024


def _norm_call(degs, x):
    return pl.pallas_call(
        _norm_body,
        grid=(_NPAD // _BN,),
        in_specs=[
            pl.BlockSpec((_NC, _BN, 16), lambda i: (0, i, 0)),
            pl.BlockSpec((_BN, _D), lambda i: (i, 0)),
        ],
        out_specs=[
            pl.BlockSpec((_BN, _D), lambda i: (i, 0)),
            pl.BlockSpec((_BN, 1), lambda i: (i, 0)),
            pl.BlockSpec((_BN, 1), lambda i: (i, 0)),
        ],
        out_shape=[
            jax.ShapeDtypeStruct((_NPAD, _D), jnp.float32),
            jax.ShapeDtypeStruct((_NPAD, 1), jnp.float32),
            jax.ShapeDtypeStruct((_NPAD, 1), jnp.float32),
        ],
    )(degs, x)


def _layer_call(p, n_in, n_out, wm, bm, relu):
    return pl.pallas_call(
        functools.partial(_layer_body, relu=relu),
        grid=(_NPAD // _BN,),
        in_specs=[
            pl.BlockSpec((_NC, _BN, _D), lambda i: (0, i, 0)),
            pl.BlockSpec((_BN, 1), lambda i: (i, 0)),
            pl.BlockSpec((_BN, 1), lambda i: (i, 0)),
            pl.BlockSpec((_D, _D), lambda i: (0, 0)),
            pl.BlockSpec((1, _D), lambda i: (0, 0)),
        ],
        out_specs=pl.BlockSpec((_BN, _D), lambda i: (i, 0)),
        out_shape=jax.ShapeDtypeStruct((_NPAD, _D), jnp.float32),
    )(p, n_in, n_out, wm, bm)


def kernel(x, edge_index, W1, b1, W2, b2):
    pad = _EPAD - _E
    padnode = _N + (jnp.arange(pad, dtype=jnp.int32) % (_NPAD - _N))
    ei = jnp.concatenate(
        [edge_index, jnp.stack([padnode, padnode])], axis=1)
    epk = (ei[0] | (ei[1] << 14)).reshape(_NW, _NCHT, _CH)
    x_pad = jnp.pad(x, ((0, _NPAD - _N), (0, 0)))

    degs = _deg_kernel(epk)
    h1, n_in, n_out = _norm_call(degs, x_pad)
    p1 = _agg_kernel(h1, epk)
    h2 = _layer_call(p1, n_in, n_out, W1, b1.reshape(1, _D), relu=True)
    p2 = _agg_kernel(h2, epk)
    out = _layer_call(p2, n_in, n_out, W2, b2.reshape(1, _D), relu=False)
    return out[:_N]


# speedup vs baseline: 4.5925x; 1.1301x over previous
import functools

import jax
import jax.numpy as jnp
from jax import lax
from jax.experimental import pallas as pl
from jax.experimental.pallas import tpu as pltpu, tpu_sc as plsc

_N = 10000
_E = 320000
_D = 128

_MESH = plsc.VectorSubcoreMesh(core_axis_name="c", subcore_axis_name="s")
_NC = _MESH.num_cores
_NS = _MESH.num_subcores
_NW = _NC * _NS
_CH = 64
_NCHT = 160
_EPT = _NCHT * _CH
_EPAD = _NW * _EPT
_NBUF = 3
_NITER = _NCHT // _NBUF
_NREM = _NCHT - _NITER * _NBUF
_NPAD = 10240
_ZROWS = _NPAD // _NS


def _unpack_edges(src_v, dst_v):
    def urow(r, _):
        def ucol(c, _):
            v = src_v[r, pl.ds(c * 16, 16)]
            dst_v[r, pl.ds(c * 16, 16)] = lax.shift_right_logical(v, 14)
            src_v[r, pl.ds(c * 16, 16)] = v & 0x3FFF
            return 0
        return lax.fori_loop(0, _CH // 16, ucol, 0)
    lax.fori_loop(0, _NCHT, urow, 0)


@functools.partial(
    pl.kernel,
    out_type=jax.ShapeDtypeStruct((_NC, _NPAD, 16), jnp.float32),
    mesh=_MESH,
    compiler_params=pltpu.CompilerParams(use_tc_tiling_on_sc=False),
    scratch_types=[
        pltpu.VMEM((_NCHT, _CH), jnp.int32),
        pltpu.VMEM((_NCHT, _CH), jnp.int32),
        pltpu.VMEM((_CH, 16), jnp.float32),
        pltpu.VMEM((_CH, 16), jnp.float32),
        pltpu.VMEM_SHARED((_NPAD, 16), jnp.float32),
        pltpu.SemaphoreType.DMA,
        pltpu.SemaphoreType.DMA,
    ],
)
def _deg_kernel(epk, out, src_v, dst_v, po, pi, table, s0, s1):
    cid = lax.axis_index("c")
    sid = lax.axis_index("s")
    pltpu.sync_copy(epk.at[cid, sid], src_v)

    zero16 = jnp.zeros((16,), jnp.float32)

    def zfill(i, _):
        po[i, pl.ds(0, 16)] = zero16
        return 0

    lax.fori_loop(0, _CH, zfill, 0)
    for z in range(_ZROWS // _CH):
        pltpu.sync_copy(po, table.at[pl.ds(sid * _ZROWS + z * _CH, _CH)])

    lane = lax.iota(jnp.int32, 16)
    po_vec = jnp.where(lane < 8, 1.0, 0.0).astype(jnp.float32)
    pi_vec = jnp.where(lane >= 8, 1.0, 0.0).astype(jnp.float32)

    def fill(i, _):
        po[i, pl.ds(0, 16)] = po_vec
        pi[i, pl.ds(0, 16)] = pi_vec
        return 0

    lax.fori_loop(0, _CH, fill, 0)
    _unpack_edges(src_v, dst_v)
    plsc.subcore_barrier()

    def chunk(j, _):
        @pl.when(j >= 2)
        def _():
            pltpu.make_async_copy(po, table.at[src_v.at[j - 2]], s0).wait()
            pltpu.make_async_copy(pi, table.at[dst_v.at[j - 2]], s1).wait()

        pltpu.async_copy(po, table.at[src_v.at[j]], s0, add=True)
        pltpu.async_copy(pi, table.at[dst_v.at[j]], s1, add=True)
        return 0

    lax.fori_loop(0, _NCHT, chunk, 0)
    for jl in (_NCHT - 2, _NCHT - 1):
        pltpu.make_async_copy(po, table.at[src_v.at[jl]], s0).wait()
        pltpu.make_async_copy(pi, table.at[dst_v.at[jl]], s1).wait()
    plsc.subcore_barrier()
    pltpu.sync_copy(
        table.at[pl.ds(sid * _ZROWS, _ZROWS)],
        out.at[cid, pl.ds(sid * _ZROWS, _ZROWS)],
    )


@functools.partial(
    pl.kernel,
    out_type=jax.ShapeDtypeStruct((_NC, _NPAD, _D), jnp.float32),
    mesh=_MESH,
    compiler_params=pltpu.CompilerParams(use_tc_tiling_on_sc=False),
    scratch_types=[
        pltpu.VMEM((_NCHT, _CH), jnp.int32),
        pltpu.VMEM((_NCHT, _CH), jnp.int32),
        pltpu.VMEM((_NBUF, _CH, _D), jnp.float32),
        pltpu.VMEM_SHARED((_NPAD, _D), jnp.float32),
        pltpu.SemaphoreType.DMA,
        pltpu.SemaphoreType.DMA,
        pltpu.SemaphoreType.DMA,
    ],
)
def _agg_kernel(h, epk, out, src_v, dst_v, bufs, acc, g0, g1, g2):
    gs = (g0, g1, g2)
    cid = lax.axis_index("c")
    sid = lax.axis_index("s")
    pltpu.sync_copy(epk.at[cid, sid], src_v)

    zero16 = jnp.zeros((16,), jnp.float32)

    def zrow(i, _):
        def zcol(c, _):
            bufs[0, i, pl.ds(c * 16, 16)] = zero16
            return 0
        return lax.fori_loop(0, _D // 16, zcol, 0)

    lax.fori_loop(0, _CH, zrow, 0)
    for z in range(_ZROWS // _CH):
        pltpu.sync_copy(
            bufs.at[0], acc.at[pl.ds(sid * _ZROWS + z * _CH, _CH)])

    _unpack_edges(src_v, dst_v)
    plsc.subcore_barrier()

    for k in range(_NBUF):
        pltpu.async_copy(h.at[src_v.at[k]], bufs.at[k], gs[k])

    def body(i, _):
        for k in range(_NBUF):
            j = _NBUF * i + k
            pltpu.make_async_copy(
                h.at[src_v.at[j]], bufs.at[k], gs[k]).wait()
            pltpu.sync_copy(bufs.at[k], acc.at[dst_v.at[j]], add=True)

            @pl.when(j + _NBUF < _NCHT)
            def _():
                pltpu.async_copy(
                    h.at[src_v.at[j + _NBUF]], bufs.at[k], gs[k])

        return 0

    lax.fori_loop(0, _NITER, body, 0)
    for r in range(_NREM):
        jr = _NITER * _NBUF + r
        pltpu.make_async_copy(
            h.at[src_v.at[jr]], bufs.at[r], gs[r]).wait()
        pltpu.sync_copy(bufs.at[r], acc.at[dst_v.at[jr]], add=True)
    plsc.subcore_barrier()
    pltpu.sync_copy(
        acc.at[pl.ds(sid * _ZROWS, _ZROWS)],
        out.at[cid, pl.ds(sid * _ZROWS, _ZROWS)],
    )


def _norm_body(degs_ref, x_ref, h_ref, nin_ref, nout_ref):
    d_out = degs_ref[0, :, 0:1] + degs_ref[1, :, 0:1]
    d_in = degs_ref[0, :, 8:9] + degs_ref[1, :, 8:9]
    n_out = jnp.where(d_out > 0, lax.rsqrt(jnp.maximum(d_out, 1.0)), 0.0)
    n_in = jnp.where(d_in > 0, lax.rsqrt(jnp.maximum(d_in, 1.0)), 0.0)
    h_ref[...] = x_ref[...] * n_out
    nin_ref[...] = n_in
    nout_ref[...] = n_out


def _layer_body(p_ref, nin_ref, nout_ref, w_ref, b_ref, o_ref, *, relu):
    a = (p_ref[0] + p_ref[1]) * nin_ref[...]
    o = jnp.dot(a, w_ref[...], preferred_element_type=jnp.float32) + b_ref[...]
    if relu:
        o = jnp.maximum(o, 0.0) * nout_ref[...]
    o_ref[...] = o


_BN = 1024


def _norm_call(degs, x):
    return pl.pallas_call(
        _norm_body,
        grid=(_NPAD // _BN,),
        in_specs=[
            pl.BlockSpec((_NC, _BN, 16), lambda i: (0, i, 0)),
            pl.BlockSpec((_BN, _D), lambda i: (i, 0)),
        ],
        out_specs=[
            pl.BlockSpec((_BN, _D), lambda i: (i, 0)),
            pl.BlockSpec((_BN, 1), lambda i: (i, 0)),
            pl.BlockSpec((_BN, 1), lambda i: (i, 0)),
        ],
        out_shape=[
            jax.ShapeDtypeStruct((_NPAD, _D), jnp.float32),
            jax.ShapeDtypeStruct((_NPAD, 1), jnp.float32),
            jax.ShapeDtypeStruct((_NPAD, 1), jnp.float32),
        ],
    )(degs, x)


def _layer_call(p, n_in, n_out, wm, bm, relu):
    return pl.pallas_call(
        functools.partial(_layer_body, relu=relu),
        grid=(_NPAD // _BN,),
        in_specs=[
            pl.BlockSpec((_NC, _BN, _D), lambda i: (0, i, 0)),
            pl.BlockSpec((_BN, 1), lambda i: (i, 0)),
            pl.BlockSpec((_BN, 1), lambda i: (i, 0)),
            pl.BlockSpec((_D, _D), lambda i: (0, 0)),
            pl.BlockSpec((1, _D), lambda i: (0, 0)),
        ],
        out_specs=pl.BlockSpec((_BN, _D), lambda i: (i, 0)),
        out_shape=jax.ShapeDtypeStruct((_NPAD, _D), jnp.float32),
    )(p, n_in, n_out, wm, bm)


def kernel(x, edge_index, W1, b1, W2, b2):
    pad = _EPAD - _E
    padnode = _N + (jnp.arange(pad, dtype=jnp.int32) % (_NPAD - _N))
    ei = jnp.concatenate(
        [edge_index, jnp.stack([padnode, padnode])], axis=1)
    epk = (ei[0] | (ei[1] << 14)).reshape(_NC, _NS, _NCHT, _CH)
    x_pad = jnp.pad(x, ((0, _NPAD - _N), (0, 0)))

    degs = _deg_kernel(epk)
    h1, n_in, n_out = _norm_call(degs, x_pad)
    p1 = _agg_kernel(h1, epk)
    h2 = _layer_call(p1, n_in, n_out, W1, b1.reshape(1, _D), relu=True)
    p2 = _agg_kernel(h2, epk)
    out = _layer_call(p2, n_in, n_out, W2, b2.reshape(1, _D), relu=False)
    return out[:_N]
